# baseline (speedup 1.0000x reference)
import jax
import jax.numpy as jnp
from jax import lax
from jax.experimental import pallas as pl
from jax.experimental.pallas import tpu as pltpu
from jax.experimental.pallas import tpu_sc as plsc
import numpy as np

ROWS = 128
N = 32768
L = 16
C = N // L
BINS = 256
G = 4
NC, NS = 2, 16
NW = NC * NS
RPW = ROWS // NW

_MININT = np.int32(-(2 ** 31))
_ALLONES = np.int32(-1)


def _digit(k, p):
    if p == 0:
        s = k
    else:
        s = lax.shift_right_logical(k, jnp.full((L,), 8 * p, jnp.int32))
    return jnp.bitwise_and(s, jnp.full((L,), 255, jnp.int32))


def _to_key(b):
    return jnp.where(b < 0, jnp.bitwise_xor(b, _ALLONES),
                     jnp.bitwise_xor(b, _MININT))


def _from_key(k):
    return jnp.where(k < 0, jnp.bitwise_xor(k, _MININT),
                     jnp.bitwise_xor(k, _ALLONES))


def _phi(r):
    return jnp.bitwise_or(
        lax.shift_left(jnp.bitwise_and(r, jnp.full((L,), C - 1, jnp.int32)),
                       jnp.full((L,), 4, jnp.int32)),
        lax.shift_right_logical(r, jnp.full((L,), 11, jnp.int32)))


def _sc_sort_body(xs_hbm, out_hbm, buf_f, key_a, key_b, totals, h_a, h_b):
    wid = lax.axis_index("s") * NC + lax.axis_index("c")
    lanes = lax.iota(jnp.int32, L)
    ones = jnp.full((L,), 1, jnp.int32)
    zeros = jnp.zeros((L,), jnp.int32)
    lane0 = lanes == 0

    def zero(h):
        @plsc.parallel_loop(0, BINS, unroll=4)
        def _z(j):
            h[pl.ds(j * L, L)] = zeros

    def scan(h):
        @plsc.parallel_loop(0, BINS, unroll=4)
        def _local(j):
            sl = pl.ds(j * L, L)
            v = h[sl]
            incl = plsc.cumsum(v)
            h[sl] = incl - v
            plsc.store_scatter(totals, [jnp.full((L,), j, jnp.int32)],
                               jnp.broadcast_to(incl[L - 1], (L,)),
                               mask=lane0)

        def _tot(u, carry):
            sl = pl.ds(u * L, L)
            tv = totals[sl]
            tincl = plsc.cumsum(tv)
            totals[sl] = tincl - tv + carry
            return carry + jnp.sum(tv)
        lax.fori_loop(0, BINS // L, _tot, jnp.int32(0))

        @plsc.parallel_loop(0, BINS, unroll=4)
        def _add(j):
            sl = pl.ds(j * L, L)
            base = plsc.load_gather(totals, [jnp.full((L,), j, jnp.int32)])
            h[sl] = h[sl] + base

    def do_row(rr, _carry):
        row = wid * RPW + rr
        pltpu.sync_copy(xs_hbm.at[row], buf_f)

        zero(h_a)

        @plsc.parallel_loop(0, C, unroll=8)
        def _hist0(i):
            b = buf_f[pl.ds(i * L, L)]
            idx = _digit(_to_key(b), 0) * L + lanes
            plsc.addupdate_scatter(h_a, [idx], ones)

        for p in range(4):
            src, dst = [(buf_f, key_a), (key_a, key_b),
                        (key_b, key_a), (key_a, buf_f)][p]
            h, hn = (h_a, h_b) if p % 2 == 0 else (h_b, h_a)

            scan(h)
            if p < 3:
                zero(hn)

            def perm_body(g, _, p=p, src=src, dst=dst, h=h, hn=hn):
                base = g * G
                ks = []
                for j in range(G):
                    v = src[pl.ds((base + j) * L, L)]
                    ks.append(_to_key(v) if p == 0 else v)
                idxs = [_digit(k, p) * L + lanes for k in ks]
                rs = [plsc.load_gather(h, [idx]) for idx in idxs]
                for j in range(G):
                    c = rs[j]
                    for jp in range(j):
                        c = c + jnp.where(idxs[jp] == idxs[j], ones, zeros)
                    rs[j] = c
                for idx in idxs:
                    plsc.addupdate_scatter(h, [idx], ones)
                for j in range(G):
                    if p < 3:
                        plsc.store_scatter(dst, [_phi(rs[j])], ks[j])
                        cell2 = jnp.bitwise_or(
                            lax.shift_left(_digit(ks[j], p + 1),
                                           jnp.full((L,), 4, jnp.int32)),
                            lax.shift_right_logical(
                                rs[j], jnp.full((L,), 11, jnp.int32)))
                        plsc.addupdate_scatter(hn, [cell2], ones)
                    else:
                        plsc.store_scatter(dst, [rs[j]], _from_key(ks[j]))
                return 0
            lax.fori_loop(0, C // G, perm_body, 0, unroll=2)

        pltpu.sync_copy(buf_f, out_hbm.at[row])
        return 0

    lax.fori_loop(0, RPW, do_row, 0)


_sc_sort = pl.kernel(
    _sc_sort_body,
    out_type=jax.ShapeDtypeStruct((ROWS, N), jnp.int32),
    mesh=plsc.VectorSubcoreMesh(core_axis_name="c", subcore_axis_name="s"),
    compiler_params=pltpu.CompilerParams(needs_layout_passes=False),
    scratch_types=[
        pltpu.VMEM((N,), jnp.int32),
        pltpu.VMEM((N,), jnp.int32),
        pltpu.VMEM((N,), jnp.int32),
        pltpu.VMEM((BINS,), jnp.int32),
        pltpu.VMEM((BINS * L,), jnp.int32),
        pltpu.VMEM((BINS * L,), jnp.int32),
    ],
)


def kernel(xs):
    xs_i = lax.bitcast_convert_type(xs, jnp.int32)
    return lax.bitcast_convert_type(_sc_sort(xs_i), jnp.float32)


# speedup vs baseline: 1.0616x; 1.0616x over previous
import jax
import jax.numpy as jnp
from jax import lax
from jax.experimental import pallas as pl
from jax.experimental.pallas import tpu as pltpu
from jax.experimental.pallas import tpu_sc as plsc
import numpy as np

ROWS = 128
N = 32768
L = 16
C = N // L
BINS = 256
G = 8
NC, NS = 2, 16
NW = NC * NS
RPW = ROWS // NW

_MININT = np.int32(-(2 ** 31))
_ALLONES = np.int32(-1)


def _digit(k, p):
    if p == 0:
        s = k
    else:
        s = lax.shift_right_logical(k, jnp.full((L,), 8 * p, jnp.int32))
    if p == 3:
        return s
    return jnp.bitwise_and(s, jnp.full((L,), 255, jnp.int32))


def _to_key(b):
    return jnp.where(b < 0, jnp.bitwise_xor(b, _ALLONES),
                     jnp.bitwise_xor(b, _MININT))


def _from_key(k):
    return jnp.where(k < 0, jnp.bitwise_xor(k, _MININT),
                     jnp.bitwise_xor(k, _ALLONES))


def _phi(r):
    return jnp.bitwise_or(
        lax.shift_left(jnp.bitwise_and(r, jnp.full((L,), C - 1, jnp.int32)),
                       jnp.full((L,), 4, jnp.int32)),
        lax.shift_right_logical(r, jnp.full((L,), 11, jnp.int32)))


def _sc_sort_body(xs_hbm, out_hbm, buf_f, key_a, key_b, totals, h_a, h_b,
                  in_sem, out_sem):
    wid = lax.axis_index("s") * NC + lax.axis_index("c")
    lanes = lax.iota(jnp.int32, L)
    ones = jnp.full((L,), 1, jnp.int32)
    zeros = jnp.zeros((L,), jnp.int32)
    lane0 = lanes == 0

    def zero(h):
        @plsc.parallel_loop(0, BINS, unroll=4)
        def _z(j):
            h[pl.ds(j * L, L)] = zeros

    def scan(h):
        @plsc.parallel_loop(0, BINS, unroll=4)
        def _local(j):
            sl = pl.ds(j * L, L)
            v = h[sl]
            incl = plsc.cumsum(v)
            h[sl] = incl - v
            plsc.store_scatter(totals, [jnp.full((L,), j, jnp.int32)],
                               jnp.broadcast_to(incl[L - 1], (L,)),
                               mask=lane0)

        def _tot(u, carry):
            sl = pl.ds(u * L, L)
            tv = totals[sl]
            tincl = plsc.cumsum(tv)
            totals[sl] = tincl - tv + carry
            return carry + jnp.sum(tv)
        lax.fori_loop(0, BINS // L, _tot, jnp.int32(0))

        @plsc.parallel_loop(0, BINS, unroll=4)
        def _add(j):
            sl = pl.ds(j * L, L)
            base = plsc.load_gather(totals, [jnp.full((L,), j, jnp.int32)])
            h[sl] = h[sl] + base

    def in_copy(row):
        return pltpu.make_async_copy(xs_hbm.at[row], buf_f, in_sem)

    def out_copy(row):
        return pltpu.make_async_copy(key_b, out_hbm.at[row], out_sem)

    in_copy(wid * RPW).start()

    def do_row(rr, _carry):
        row = wid * RPW + rr
        in_copy(row).wait()

        zero(h_a)

        @plsc.parallel_loop(0, C, unroll=8)
        def _hist0(i):
            b = buf_f[pl.ds(i * L, L)]
            idx = _digit(_to_key(b), 0) * L + lanes
            plsc.addupdate_scatter(h_a, [idx], ones)

        for p in range(4):
            src, dst = [(buf_f, key_a), (key_a, buf_f),
                        (buf_f, key_a), (key_a, key_b)][p]
            h, hn = (h_a, h_b) if p % 2 == 0 else (h_b, h_a)

            scan(h)
            if p < 3:
                zero(hn)
            if p == 3:
                @pl.when(rr + 1 < RPW)
                def _prefetch():
                    in_copy(row + 1).start()

                @pl.when(rr > 0)
                def _drain_out():
                    out_copy(row - 1).wait()

            def perm_body(g, _, p=p, src=src, dst=dst, h=h, hn=hn):
                base = g * G
                ks = []
                for j in range(G):
                    v = src[pl.ds((base + j) * L, L)]
                    ks.append(_to_key(v) if p == 0 else v)
                idxs = [_digit(k, p) * L + lanes for k in ks]
                rs = [plsc.load_gather(h, [idx]) for idx in idxs]
                for j in range(G):
                    c = rs[j]
                    for jp in range(j):
                        c = c + jnp.where(idxs[jp] == idxs[j], ones, zeros)
                    rs[j] = c
                for idx in idxs:
                    plsc.addupdate_scatter(h, [idx], ones)
                for j in range(G):
                    if p < 3:
                        plsc.store_scatter(dst, [_phi(rs[j])], ks[j])
                        cell2 = jnp.bitwise_or(
                            lax.shift_left(_digit(ks[j], p + 1),
                                           jnp.full((L,), 4, jnp.int32)),
                            lax.shift_right_logical(
                                rs[j], jnp.full((L,), 11, jnp.int32)))
                        plsc.addupdate_scatter(hn, [cell2], ones)
                    else:
                        plsc.store_scatter(dst, [rs[j]], _from_key(ks[j]))
                return 0
            lax.fori_loop(0, C // G, perm_body, 0)

        out_copy(row).start()
        return 0

    lax.fori_loop(0, RPW, do_row, 0)
    out_copy(wid * RPW + RPW - 1).wait()


_sc_sort = pl.kernel(
    _sc_sort_body,
    out_type=jax.ShapeDtypeStruct((ROWS, N), jnp.int32),
    mesh=plsc.VectorSubcoreMesh(core_axis_name="c", subcore_axis_name="s"),
    compiler_params=pltpu.CompilerParams(needs_layout_passes=False),
    scratch_types=[
        pltpu.VMEM((N,), jnp.int32),
        pltpu.VMEM((N,), jnp.int32),
        pltpu.VMEM((N,), jnp.int32),
        pltpu.VMEM((BINS,), jnp.int32),
        pltpu.VMEM((BINS * L,), jnp.int32),
        pltpu.VMEM((BINS * L,), jnp.int32),
        pltpu.SemaphoreType.DMA,
        pltpu.SemaphoreType.DMA,
    ],
)


def kernel(xs):
    xs_i = lax.bitcast_convert_type(xs, jnp.int32)
    return lax.bitcast_convert_type(_sc_sort(xs_i), jnp.float32)
